# baseline (speedup 1.0000x reference)
import functools

import jax
import jax.numpy as jnp
from jax import lax
from jax.experimental import pallas as pl
from jax.experimental.pallas import tpu as pltpu
from jax.experimental.pallas import tpu_sc as plsc

NB = 4096
NT = 200
D = 64
NW = 32
BT = 128
NBT = NB // BT

_mesh = plsc.VectorSubcoreMesh(core_axis_name="c", subcore_axis_name="s")


@functools.partial(
    pl.kernel,
    mesh=_mesh,
    compiler_params=pltpu.CompilerParams(needs_layout_passes=False),
    out_type=jax.ShapeDtypeStruct((NT, D // 8, NBT, 8, BT), jnp.float32),
    scratch_types=[
        pltpu.VMEM((BT,), jnp.int32),
        pltpu.VMEM((2, BT, D), jnp.float32),
        pltpu.VMEM((2, D // 8, 8, BT), jnp.float32),
        pltpu.SemaphoreType.DMA,
        pltpu.SemaphoreType.DMA,
    ],
)
def _emb_lookup(xt_hbm, table_hbm, out_hbm, idx_v, rows_v, tr_v,
                in_sem, out_sem):
    bt = lax.axis_index("s") * 2 + lax.axis_index("c")
    b0 = bt * BT

    lanes = lax.iota(jnp.int32, 16)

    def fetch_and_fire(t, db):
        pltpu.sync_copy(xt_hbm.at[t, pl.ds(b0, BT)], idx_v)
        for jg in range(BT // 16):
            v = idx_v[pl.ds(jg * 16, 16)]
            for l in range(16):
                pltpu.async_copy(
                    table_hbm.at[v[l]], rows_v.at[db, jg * 16 + l], in_sem
                )

    def wait_rows(db):
        pltpu.make_async_copy(
            table_hbm.at[pl.ds(0, BT)], rows_v.at[db], in_sem
        ).wait()

    def transpose(db):
        for dt in range(D // 8):
            for d_in in range(8):
                d_vec = jnp.full((16,), dt * 8 + d_in, jnp.int32)
                for bg in range(BT // 16):
                    b_vec = bg * 16 + lanes
                    v = plsc.load_gather(rows_v.at[db], [b_vec, d_vec])
                    tr_v[db, dt, d_in, pl.ds(bg * 16, 16)] = v

    def wait_out(db):
        pltpu.make_async_copy(
            out_hbm.at[0, :, 0], tr_v.at[db], out_sem
        ).wait()

    def unit(t, db):
        wait_rows(db)

        @pl.when(t + 1 < NT)
        def _():
            fetch_and_fire(t + 1, 1 - db)

        @pl.when(t >= 2)
        def _():
            wait_out(db)

        transpose(db)
        pltpu.async_copy(tr_v.at[db], out_hbm.at[t, :, bt], out_sem)

    fetch_and_fire(0, 0)

    def body(p, carry):
        unit(p * 2, 0)
        unit(p * 2 + 1, 1)
        return carry

    lax.fori_loop(0, NT // 2, body, 0)
    for db in range(2):
        wait_out(db)


def kernel(x, emb):
    out5 = _emb_lookup(x.T.astype(jnp.int32), emb)
    return jnp.transpose(out5, (2, 4, 0, 1, 3)).reshape(NB, NT, D)


# speedup vs baseline: 1.0172x; 1.0172x over previous
import functools

import jax
import jax.numpy as jnp
from jax import lax
from jax.experimental import pallas as pl
from jax.experimental.pallas import tpu as pltpu
from jax.experimental.pallas import tpu_sc as plsc

NB = 4096
NT = 200
D = 64
DP = 128
NW = 32
BT = 128
NBT = NB // BT

_mesh = plsc.VectorSubcoreMesh(core_axis_name="c", subcore_axis_name="s")


@functools.partial(
    pl.kernel,
    mesh=_mesh,
    compiler_params=pltpu.CompilerParams(needs_layout_passes=False),
    out_type=jax.ShapeDtypeStruct((NT, D // 8, NBT, 8, BT), jnp.float32),
    scratch_types=[
        pltpu.VMEM((NT, BT), jnp.int32),
        pltpu.VMEM((2, BT, DP), jnp.float32),
        pltpu.VMEM((2, D // 8, 8, BT), jnp.float32),
        pltpu.SemaphoreType.DMA,
        pltpu.SemaphoreType.DMA,
    ],
)
def _emb_lookup(xt_hbm, table_hbm, out_hbm, idx_v, rows_v, tr_v,
                in_sem, out_sem):
    bt = lax.axis_index("s") * 2 + lax.axis_index("c")
    b0 = bt * BT
    pltpu.sync_copy(xt_hbm.at[:, pl.ds(b0, BT)], idx_v)

    lanes = lax.iota(jnp.int32, 16)

    def fire_gather(t, db):
        pltpu.async_copy(table_hbm.at[idx_v.at[t]], rows_v.at[db], in_sem)

    def wait_rows(db):
        pltpu.make_async_copy(
            table_hbm.at[pl.ds(0, BT)], rows_v.at[db], in_sem
        ).wait()

    def transpose(db):
        for dt in range(D // 8):
            for d_in in range(8):
                d_vec = jnp.full((16,), dt * 8 + d_in, jnp.int32)
                for bg in range(BT // 16):
                    b_vec = bg * 16 + lanes
                    v = plsc.load_gather(rows_v.at[db], [b_vec, d_vec])
                    tr_v[db, dt, d_in, pl.ds(bg * 16, 16)] = v

    def wait_out(db):
        pltpu.make_async_copy(
            out_hbm.at[0, :, 0], tr_v.at[db], out_sem
        ).wait()

    def unit(t, db):
        wait_rows(db)

        @pl.when(t + 1 < NT)
        def _():
            fire_gather(t + 1, 1 - db)

        @pl.when(t >= 2)
        def _():
            wait_out(db)

        transpose(db)
        pltpu.async_copy(tr_v.at[db], out_hbm.at[t, :, bt], out_sem)

    fire_gather(0, 0)

    def body(p, carry):
        unit(p * 2, 0)
        unit(p * 2 + 1, 1)
        return carry

    lax.fori_loop(0, NT // 2, body, 0)
    for db in range(2):
        wait_out(db)


def kernel(x, emb):
    table = jnp.pad(emb, ((0, 0), (0, DP - D)))
    out5 = _emb_lookup(x.T.astype(jnp.int32), table)
    return jnp.transpose(out5, (2, 4, 0, 1, 3)).reshape(NB, NT, D)


# speedup vs baseline: 1.5946x; 1.5676x over previous
import functools

import jax
import jax.numpy as jnp
from jax import lax
from jax.experimental import pallas as pl
from jax.experimental.pallas import tpu as pltpu
from jax.experimental.pallas import tpu_sc as plsc

NB = 4096
NT = 200
D = 64
DP = 128
NW = 32
BT = 128
NBT = NB // BT

_mesh = plsc.VectorSubcoreMesh(core_axis_name="c", subcore_axis_name="s")


@functools.partial(
    pl.kernel,
    mesh=_mesh,
    compiler_params=pltpu.CompilerParams(needs_layout_passes=False),
    out_type=jax.ShapeDtypeStruct((NT, D // 8, NBT, 8, BT), jnp.float32),
    scratch_types=[
        pltpu.VMEM((NT, BT), jnp.int32),
        pltpu.VMEM((2, BT, DP), jnp.float32),
        pltpu.VMEM((2, D // 8, 8, BT), jnp.float32),
        pltpu.SemaphoreType.DMA,
        pltpu.SemaphoreType.DMA,
    ],
)
def _emb_lookup(xt_hbm, table_hbm, out_hbm, idx_v, rows_v, tr_v,
                in_sem, out_sem):
    bt = lax.axis_index("s") * 2 + lax.axis_index("c")
    b0 = bt * BT
    pltpu.sync_copy(xt_hbm.at[:, pl.ds(b0, BT)], idx_v)

    lanes = lax.iota(jnp.int32, 16)

    def fire_gather(t, db):
        pltpu.async_copy(table_hbm.at[idx_v.at[t]], rows_v.at[db], in_sem)

    def wait_rows(db):
        pltpu.make_async_copy(
            table_hbm.at[pl.ds(0, BT)], rows_v.at[db], in_sem
        ).wait()

    dt_vecs = [(dg * 16 + lanes) // 8 for dg in range(D // 16)]
    din_vecs = [(dg * 16 + lanes) % 8 for dg in range(D // 16)]

    def transpose(db):
        @plsc.parallel_loop(0, BT, 1, unroll=8)
        def _(b):
            bs = jnp.full((16,), b, jnp.int32)
            for dg in range(D // 16):
                v = rows_v[db, b, pl.ds(dg * 16, 16)]
                plsc.store_scatter(
                    tr_v.at[db], [dt_vecs[dg], din_vecs[dg], bs], v
                )

    def wait_out(db):
        pltpu.make_async_copy(
            out_hbm.at[0, :, 0], tr_v.at[db], out_sem
        ).wait()

    def unit(t, db):
        wait_rows(db)

        @pl.when(t + 1 < NT)
        def _():
            fire_gather(t + 1, 1 - db)

        @pl.when(t >= 2)
        def _():
            wait_out(db)

        transpose(db)
        pltpu.async_copy(tr_v.at[db], out_hbm.at[t, :, bt], out_sem)

    fire_gather(0, 0)

    def body(p, carry):
        unit(p * 2, 0)
        unit(p * 2 + 1, 1)
        return carry

    lax.fori_loop(0, NT // 2, body, 0)
    for db in range(2):
        wait_out(db)


def kernel(x, emb):
    table = jnp.pad(emb, ((0, 0), (0, DP - D)))
    out5 = _emb_lookup(x.T.astype(jnp.int32), table)
    return jnp.transpose(out5, (2, 4, 0, 1, 3)).reshape(NB, NT, D)
